# baseline (speedup 1.0000x reference)
import functools

import jax
import jax.numpy as jnp
from jax import lax
from jax.experimental import pallas as pl
from jax.experimental.pallas import tpu as pltpu
from jax.experimental.pallas import tpu_sc as plsc

_G_SA00 = 0.003
_G_SA01 = 0.001
_G_SA10 = 0.002
_G_SA11 = 3e-06

_ROWS = 131072
_COLS = 256
_K_SC = 57344
_NW = 32
_WROWS = _K_SC // _NW
_CROWS = 64
_ITERS = _WROWS // _CROWS

_TC_ROWS = _ROWS - _K_SC
_TC_BR = 4096

_GATHER_DNUMS = lax.GatherDimensionNumbers(
    offset_dims=(), collapsed_slice_dims=(0,), start_index_map=(0,))


def _inject(xbuf, pbuf, obuf):
    lane = lax.iota(jnp.int32, 16)
    lut = jnp.where(lane == 1, jnp.float32(_G_SA00), jnp.float32(0.0))
    lut = jnp.where(lane == 2, jnp.float32(_G_SA01), lut)
    lut = jnp.where(lane == 3, jnp.float32(_G_SA10), lut)
    lut = jnp.where(lane == 4, jnp.float32(_G_SA11), lut)

    def row_body(r, carry):
        for j in range(_COLS // 16):
            c = j * 16
            xv = xbuf[r, pl.ds(c, 16)]
            pv = pbuf[r, pl.ds(c, 16)]
            v = lax.gather(
                lut, pv[:, None], dimension_numbers=_GATHER_DNUMS,
                slice_sizes=(1,),
                mode=lax.GatherScatterMode.PROMISE_IN_BOUNDS)
            obuf[r, pl.ds(c, 16)] = jnp.where(pv == 0, xv, v)
        return carry

    lax.fori_loop(0, _CROWS, row_body, 0)


def _saf_sc_body(x_hbm, p_hbm, o_hbm,
                 xb0, xb1, pb0, pb1, ob0, ob1,
                 semi0, semi1, semo0, semo1):
    xb = (xb0, xb1)
    pb = (pb0, pb1)
    ob = (ob0, ob1)
    semi = (semi0, semi1)
    semo = (semo0, semo1)

    wid = lax.axis_index("s") * 2 + lax.axis_index("c")
    base = wid * _WROWS

    def row0(g):
        return pl.multiple_of(base + g * _CROWS, _CROWS)

    def start_in(g, b):
        pltpu.make_async_copy(x_hbm.at[pl.ds(row0(g), _CROWS)], xb[b], semi[b]).start()
        pltpu.make_async_copy(p_hbm.at[pl.ds(row0(g), _CROWS)], pb[b], semi[b]).start()

    def wait_in(g, b):
        pltpu.make_async_copy(x_hbm.at[pl.ds(row0(g), _CROWS)], xb[b], semi[b]).wait()
        pltpu.make_async_copy(p_hbm.at[pl.ds(row0(g), _CROWS)], pb[b], semi[b]).wait()

    def start_out(g, b):
        pltpu.make_async_copy(ob[b], o_hbm.at[pl.ds(row0(g), _CROWS)], semo[b]).start()

    def wait_out(g, b):
        pltpu.make_async_copy(ob[b], o_hbm.at[pl.ds(row0(g), _CROWS)], semo[b]).wait()

    start_in(0, 0)
    start_in(1, 1)

    for b in range(2):
        g = b
        wait_in(g, b)
        _inject(xb[b], pb[b], ob[b])
        start_in(g + 2, b)
        start_out(g, b)

    def pair_body(i, carry):
        for b in range(2):
            g = 2 * i + b
            wait_in(g, b)
            wait_out(g - 2, b)
            _inject(xb[b], pb[b], ob[b])
            start_in(g + 2, b)
            start_out(g, b)
        return carry

    lax.fori_loop(1, _ITERS // 2 - 1, pair_body, 0)

    for b in range(2):
        g = _ITERS - 2 + b
        wait_in(g, b)
        wait_out(g - 2, b)
        _inject(xb[b], pb[b], ob[b])
        start_out(g, b)

    for b in range(2):
        wait_out(_ITERS - 2 + b, b)


@functools.partial(
    pl.kernel,
    out_type=jax.ShapeDtypeStruct((_K_SC, _COLS), jnp.float32),
    mesh=plsc.VectorSubcoreMesh(core_axis_name="c", subcore_axis_name="s"),
    scratch_types=[
        pltpu.VMEM((_CROWS, _COLS), jnp.float32),
        pltpu.VMEM((_CROWS, _COLS), jnp.float32),
        pltpu.VMEM((_CROWS, _COLS), jnp.int32),
        pltpu.VMEM((_CROWS, _COLS), jnp.int32),
        pltpu.VMEM((_CROWS, _COLS), jnp.float32),
        pltpu.VMEM((_CROWS, _COLS), jnp.float32),
        pltpu.SemaphoreType.DMA,
        pltpu.SemaphoreType.DMA,
        pltpu.SemaphoreType.DMA,
        pltpu.SemaphoreType.DMA,
    ],
)
def _saf_sc(*refs):
    _saf_sc_body(*refs)


def _saf_tc_block(x_ref, p_ref, o_ref):
    x = x_ref[...]
    p = p_ref[...]
    lo = jnp.where(p == 1, jnp.float32(_G_SA00), jnp.float32(_G_SA01))
    hi = jnp.where(p == 3, jnp.float32(_G_SA10), jnp.float32(_G_SA11))
    v = jnp.where(p <= 2, lo, hi)
    o_ref[...] = jnp.where(p == 0, x, v)


def _saf_tc_tail(x, p):
    off = _K_SC // _TC_BR
    return pl.pallas_call(
        _saf_tc_block,
        grid=(_TC_ROWS // _TC_BR,),
        in_specs=[
            pl.BlockSpec((_TC_BR, _COLS), lambda i: (i + off, 0)),
            pl.BlockSpec((_TC_BR, _COLS), lambda i: (i + off, 0)),
        ],
        out_specs=pl.BlockSpec((_TC_BR, _COLS), lambda i: (i, 0)),
        out_shape=jax.ShapeDtypeStruct((_TC_ROWS, _COLS), jnp.float32),
    )(x, p)


def kernel(input, p_state):
    x = jnp.transpose(input, (0, 2, 3, 1)).reshape(_ROWS, _COLS)
    p = jnp.transpose(p_state, (0, 2, 3, 1)).reshape(_ROWS, _COLS)
    head = _saf_sc(x, p)
    tail = _saf_tc_tail(x, p)
    out = jnp.concatenate([head, tail], axis=0)
    return jnp.transpose(out.reshape(128, 32, 32, 256), (0, 3, 1, 2))


# speedup vs baseline: 1.4508x; 1.4508x over previous
import functools

import jax
import jax.numpy as jnp
from jax import lax
from jax.experimental import pallas as pl
from jax.experimental.pallas import tpu as pltpu
from jax.experimental.pallas import tpu_sc as plsc

_G_SA00 = 0.003
_G_SA01 = 0.001
_G_SA10 = 0.002
_G_SA11 = 3e-06

_ROWS = 131072
_COLS = 256
_K_SC = 131072
_NW = 32
_WROWS = _K_SC // _NW
_CROWS = 32
_DEPTH = 4
_ITERS = _WROWS // _CROWS

_TC_ROWS = _ROWS - _K_SC
_TC_BR = 4096

_GATHER_DNUMS = lax.GatherDimensionNumbers(
    offset_dims=(), collapsed_slice_dims=(0,), start_index_map=(0,))


def _inject(xbuf, pbuf, obuf):
    lane = lax.iota(jnp.int32, 16)
    lut = jnp.where(lane == 1, jnp.float32(_G_SA00), jnp.float32(0.0))
    lut = jnp.where(lane == 2, jnp.float32(_G_SA01), lut)
    lut = jnp.where(lane == 3, jnp.float32(_G_SA10), lut)
    lut = jnp.where(lane == 4, jnp.float32(_G_SA11), lut)

    def row_body(r, carry):
        for j in range(_COLS // 16):
            c = j * 16
            xv = xbuf[r, pl.ds(c, 16)]
            pv = pbuf[r, pl.ds(c, 16)]
            v = lax.gather(
                lut, pv[:, None], dimension_numbers=_GATHER_DNUMS,
                slice_sizes=(1,),
                mode=lax.GatherScatterMode.PROMISE_IN_BOUNDS)
            obuf[r, pl.ds(c, 16)] = jnp.where(pv == 0, xv, v)
        return carry

    lax.fori_loop(0, _CROWS, row_body, 0)


def _saf_sc_body(x_hbm, p_hbm, o_hbm, *scr):
    xb = scr[0:_DEPTH]
    pb = scr[_DEPTH:2 * _DEPTH]
    ob = scr[2 * _DEPTH:3 * _DEPTH]
    semi = scr[3 * _DEPTH:4 * _DEPTH]
    semo = scr[4 * _DEPTH:5 * _DEPTH]

    wid = lax.axis_index("s") * 2 + lax.axis_index("c")
    base = wid * _WROWS

    def row0(g):
        return pl.multiple_of(base + g * _CROWS, _CROWS)

    def start_in(g, b):
        pltpu.make_async_copy(x_hbm.at[pl.ds(row0(g), _CROWS)], xb[b], semi[b]).start()
        pltpu.make_async_copy(p_hbm.at[pl.ds(row0(g), _CROWS)], pb[b], semi[b]).start()

    def wait_in(g, b):
        pltpu.make_async_copy(x_hbm.at[pl.ds(row0(g), _CROWS)], xb[b], semi[b]).wait()
        pltpu.make_async_copy(p_hbm.at[pl.ds(row0(g), _CROWS)], pb[b], semi[b]).wait()

    def start_out(g, b):
        pltpu.make_async_copy(ob[b], o_hbm.at[pl.ds(row0(g), _CROWS)], semo[b]).start()

    def wait_out(g, b):
        pltpu.make_async_copy(ob[b], o_hbm.at[pl.ds(row0(g), _CROWS)], semo[b]).wait()

    for b in range(_DEPTH):
        start_in(b, b)

    for b in range(_DEPTH):
        g = b
        wait_in(g, b)
        _inject(xb[b], pb[b], ob[b])
        start_in(g + _DEPTH, b)
        start_out(g, b)

    def group_body(i, carry):
        for b in range(_DEPTH):
            g = _DEPTH * i + b
            wait_in(g, b)
            wait_out(g - _DEPTH, b)
            _inject(xb[b], pb[b], ob[b])
            start_in(g + _DEPTH, b)
            start_out(g, b)
        return carry

    lax.fori_loop(1, _ITERS // _DEPTH - 1, group_body, 0)

    for b in range(_DEPTH):
        g = _ITERS - _DEPTH + b
        wait_in(g, b)
        wait_out(g - _DEPTH, b)
        _inject(xb[b], pb[b], ob[b])
        start_out(g, b)

    for b in range(_DEPTH):
        wait_out(_ITERS - _DEPTH + b, b)


@functools.partial(
    pl.kernel,
    out_type=jax.ShapeDtypeStruct((_K_SC, _COLS), jnp.float32),
    mesh=plsc.VectorSubcoreMesh(core_axis_name="c", subcore_axis_name="s"),
    scratch_types=(
        [pltpu.VMEM((_CROWS, _COLS), jnp.float32)] * _DEPTH
        + [pltpu.VMEM((_CROWS, _COLS), jnp.int32)] * _DEPTH
        + [pltpu.VMEM((_CROWS, _COLS), jnp.float32)] * _DEPTH
        + [pltpu.SemaphoreType.DMA] * (2 * _DEPTH)
    ),
)
def _saf_sc(*refs):
    _saf_sc_body(*refs)


def _saf_tc_block(x_ref, p_ref, o_ref):
    x = x_ref[...]
    p = p_ref[...]
    lo = jnp.where(p == 1, jnp.float32(_G_SA00), jnp.float32(_G_SA01))
    hi = jnp.where(p == 3, jnp.float32(_G_SA10), jnp.float32(_G_SA11))
    v = jnp.where(p <= 2, lo, hi)
    o_ref[...] = jnp.where(p == 0, x, v)


def _saf_tc_tail(x, p):
    off = _K_SC // _TC_BR
    return pl.pallas_call(
        _saf_tc_block,
        grid=(_TC_ROWS // _TC_BR,),
        in_specs=[
            pl.BlockSpec((_TC_BR, _COLS), lambda i: (i + off, 0)),
            pl.BlockSpec((_TC_BR, _COLS), lambda i: (i + off, 0)),
        ],
        out_specs=pl.BlockSpec((_TC_BR, _COLS), lambda i: (i, 0)),
        out_shape=jax.ShapeDtypeStruct((_TC_ROWS, _COLS), jnp.float32),
    )(x, p)


def kernel(input, p_state):
    x = jnp.transpose(input, (0, 2, 3, 1)).reshape(_ROWS, _COLS)
    p = jnp.transpose(p_state, (0, 2, 3, 1)).reshape(_ROWS, _COLS)
    out = _saf_sc(x, p)
    return jnp.transpose(out.reshape(128, 32, 32, 256), (0, 3, 1, 2))
